# baseline (speedup 1.0000x reference)
import functools

import jax
import jax.numpy as jnp
from jax import lax
from jax.experimental import pallas as pl
from jax.experimental.pallas import tpu as pltpu
from jax.experimental.pallas import tpu_sc as plsc

_D = 64
_NC, _NS = 2, 16
_NW = _NC * _NS
_TB = 128
_NBUF = 4
_WBUF = 2


@functools.cache
def _make_lookup(T, R):
    t_per_w = T // _NW
    nb_t = t_per_w // _TB
    n = R * nb_t
    assert T % (_NW * _TB) == 0 and nb_t == 4 and n % 4 == 0

    def body(idx_hbm, table_hbm, out_hbm, idx_v, idxl, rows, outb, gsem, wsem):
        wid = lax.axis_index("s") * _NC + lax.axis_index("c")
        tw0 = wid * t_per_w
        iota = lax.iota(jnp.int32, 16)

        pltpu.sync_copy(idx_hbm.at[pl.ds(tw0 * R, t_per_w * R)], idx_v)

        def build_list(b, buf):
            r = b // nb_t
            tb = b % nb_t
            for k in range(_TB // 16):
                src = (tb * _TB + k * 16 + iota) * R + r
                idxl[buf, pl.ds(k * 16, 16)] = plsc.load_gather(idx_v, [src])

        def g_start(buf):
            pltpu.async_copy(table_hbm.at[idxl.at[buf]], rows.at[buf],
                             gsem.at[buf])

        def g_wait(buf):
            pltpu.make_async_copy(table_hbm.at[idxl.at[buf]], rows.at[buf],
                                  gsem.at[buf]).wait()

        def out_slice(b):
            r = b // nb_t
            tb = b % nb_t
            return out_hbm.at[r, :, pl.ds(tw0 + tb * _TB, _TB)]

        def w_start(b, buf):
            pltpu.async_copy(outb.at[buf], out_slice(b), wsem.at[buf])

        def w_wait(b, buf):
            pltpu.make_async_copy(outb.at[buf], out_slice(b),
                                  wsem.at[buf]).wait()

        c_const = [jnp.full((16,), c, jnp.int32) for c in range(_D)]

        def tp_block(sbuf, dbuf):
            def tp(m, carry):
                tt = m * 16
                tvec = tt + iota
                for c in range(_D):
                    v = plsc.load_gather(rows.at[sbuf], [tvec, c_const[c]])
                    outb[dbuf, c, pl.ds(tt, 16)] = v
                return carry
            lax.fori_loop(0, _TB // 16, tp, 0)

        for b in range(_NBUF - 1):
            build_list(b, b)
            g_start(b)

        def blk(i, carry):
            for u in range(4):
                b = i * 4 + u
                nxt = b + _NBUF - 1

                @pl.when(nxt < n)
                def _():
                    build_list(nxt, (u + _NBUF - 1) % _NBUF)
                    g_start((u + _NBUF - 1) % _NBUF)

                g_wait(u)

                @pl.when(b >= _WBUF)
                def _():
                    w_wait(b - _WBUF, u % _WBUF)

                tp_block(u, u % _WBUF)
                w_start(b, u % _WBUF)
            return carry

        lax.fori_loop(0, n // 4, blk, 0)
        w_wait(n - 2, (n - 2) % _WBUF)
        w_wait(n - 1, (n - 1) % _WBUF)

    return pl.kernel(
        body,
        mesh=plsc.VectorSubcoreMesh(core_axis_name="c", subcore_axis_name="s"),
        compiler_params=pltpu.CompilerParams(use_tc_tiling_on_sc=False,
                                             needs_layout_passes=False),
        out_type=jax.ShapeDtypeStruct((R, _D, T), jnp.float32),
        scratch_types=[
            pltpu.VMEM((T // _NW * R,), jnp.int32),
            pltpu.VMEM((_NBUF, _TB), jnp.int32),
            pltpu.VMEM((_NBUF, _TB, _D), jnp.float32),
            pltpu.VMEM((_WBUF, _D, _TB), jnp.float32),
            pltpu.SemaphoreType.DMA((_NBUF,)),
            pltpu.SemaphoreType.DMA((_WBUF,)),
        ],
    )


def kernel(token_ids, weight):
    T, R = token_ids.shape
    flat = token_ids.reshape(-1).astype(jnp.int32)
    out3 = _make_lookup(T, R)(flat, weight)
    return jnp.transpose(out3, (2, 0, 1))


# speedup vs baseline: 1.1440x; 1.1440x over previous
import functools

import jax
import jax.numpy as jnp
from jax import lax
from jax.experimental import pallas as pl
from jax.experimental.pallas import tpu as pltpu
from jax.experimental.pallas import tpu_sc as plsc

_D = 64
_NC, _NS = 2, 16
_NW = _NC * _NS
_TB = 128
_NBUF = 4
_WBUF = 2


@functools.cache
def _make_transpose(V):
    n_slab = V // _TB
    tail = V % _TB
    base, rem = divmod(n_slab, _NW)
    trip = base + (1 if rem else 0)
    assert tail % 8 == 0 and _D == 64

    def body(w3_hbm, wt_hbm, flat_hbm, slab, wbuf0, wbuf1, rsem, wsem):
        wid = lax.axis_index("s") * _NC + lax.axis_index("c")
        n_w = base + jnp.where(wid < rem, 1, 0)
        start = wid * base + jnp.minimum(wid, rem)
        iota = lax.iota(jnp.int32, 16)
        iota64 = iota * _D

        def r_start(k, b):
            j0 = (start + k) * _TB
            for g in range(8):
                pltpu.async_copy(w3_hbm.at[pl.ds(g * 8, 8), pl.ds(j0, _TB)],
                                 slab.at[b, g], rsem.at[b])

        def r_wait(k, b):
            j0 = (start + k) * _TB
            for g in range(8):
                pltpu.make_async_copy(
                    w3_hbm.at[pl.ds(g * 8, 8), pl.ds(j0, _TB)],
                    slab.at[b, g], rsem.at[b]).wait()

        wbufs = [wbuf0, wbuf1]

        def tp(sb, db):
            @plsc.parallel_loop(0, (_TB // 16) * _D, unroll=8)
            def _(i):
                m = lax.shift_right_logical(i, 6)
                c = lax.bitwise_and(i, _D - 1)
                g = lax.shift_right_logical(c, 3)
                s = lax.bitwise_and(c, 7)
                v = slab[sb, g, s, pl.ds(m * 16, 16)]
                plsc.store_scatter(wbufs[db],
                                   [iota64 + (m * 16 * _D + c)], v)

        def w_start(k, b):
            j0 = (start + k) * _TB
            pltpu.async_copy(wbufs[b],
                             flat_hbm.at[pl.ds(j0 * _D, _TB * _D)],
                             wsem.at[b])

        def w_wait(k, b):
            j0 = (start + k) * _TB
            pltpu.make_async_copy(wbufs[b],
                                  flat_hbm.at[pl.ds(j0 * _D, _TB * _D)],
                                  wsem.at[b]).wait()

        r_start(0, 0)

        def blk(i, carry):
            for u in range(2):
                k = i * 2 + u

                @pl.when(k + 1 < n_w)
                def _():
                    r_start(k + 1, (u + 1) % 2)

                @pl.when(k < n_w)
                def _():
                    r_wait(k, u)

                    @pl.when(k >= 2)
                    def _():
                        w_wait(k - 2, u)

                    tp(u, u)
                    w_start(k, u)
            return carry

        lax.fori_loop(0, (trip + 1) // 2, blk, 0)

        @pl.when(n_w % 2 == 0)
        def _():
            w_wait(n_w - 2, 0)
            w_wait(n_w - 1, 1)

        @pl.when(n_w % 2 == 1)
        def _():
            w_wait(n_w - 2, 1)
            w_wait(n_w - 1, 0)

        if tail:
            @pl.when(wid == _NW - 1)
            def _():
                j0 = n_slab * _TB
                for g in range(8):
                    pltpu.sync_copy(wt_hbm.at[pl.ds(g * 8, 8), :],
                                    slab.at[0, g])
                for m in range(tail // 16):
                    for c in range(_D):
                        v = slab[0, c // 8, c % 8, pl.ds(m * 16, 16)]
                        plsc.store_scatter(wbuf0,
                                           [iota64 + (m * 16 * _D + c)], v)
                pltpu.sync_copy(wbuf0.at[pl.ds(0, tail * _D)],
                                flat_hbm.at[pl.ds(j0 * _D, tail * _D)])

    return pl.kernel(
        body,
        mesh=plsc.VectorSubcoreMesh(core_axis_name="c", subcore_axis_name="s"),
        compiler_params=pltpu.CompilerParams(use_tc_tiling_on_sc=True,
                                             needs_layout_passes=False),
        out_type=jax.ShapeDtypeStruct((V * _D,), jnp.float32),
        scratch_types=[
            pltpu.VMEM((2, 8, 8, _TB), jnp.float32),
            pltpu.VMEM((_TB * _D,), jnp.float32),
            pltpu.VMEM((_TB * _D,), jnp.float32),
            pltpu.SemaphoreType.DMA((2,)),
            pltpu.SemaphoreType.DMA((2,)),
        ],
    )


@functools.cache
def _make_lookup(T, R):
    t_per_w = T // _NW
    nb_t = t_per_w // _TB
    n = R * nb_t
    assert T % (_NW * _TB) == 0 and nb_t == 4 and n % 4 == 0

    def body(idx_hbm, table_hbm, out_hbm, idx_v, idxl, rows, outb, gsem, wsem):
        wid = lax.axis_index("s") * _NC + lax.axis_index("c")
        tw0 = wid * t_per_w
        iota = lax.iota(jnp.int32, 16)

        pltpu.sync_copy(idx_hbm.at[pl.ds(tw0 * R, t_per_w * R)], idx_v)

        def build_list(b, buf):
            r = b // nb_t
            tb = b % nb_t
            for k in range(_TB // 16):
                src = (tb * _TB + k * 16 + iota) * R + r
                idxl[buf, pl.ds(k * 16, 16)] = plsc.load_gather(idx_v, [src])

        def g_start(buf):
            pltpu.async_copy(table_hbm.at[idxl.at[buf]], rows.at[buf],
                             gsem.at[buf])

        def g_wait(buf):
            pltpu.make_async_copy(table_hbm.at[idxl.at[buf]], rows.at[buf],
                                  gsem.at[buf]).wait()

        def out_slice(b):
            r = b // nb_t
            tb = b % nb_t
            return out_hbm.at[r, :, pl.ds(tw0 + tb * _TB, _TB)]

        def w_start(b, buf):
            pltpu.async_copy(outb.at[buf], out_slice(b), wsem.at[buf])

        def w_wait(b, buf):
            pltpu.make_async_copy(outb.at[buf], out_slice(b),
                                  wsem.at[buf]).wait()

        zeros16 = jnp.zeros((16,), jnp.int32)
        ciota = [c0 + iota for c0 in range(0, _D, 16)]

        def tp_block(sbuf, dbuf):
            @plsc.parallel_loop(0, _TB, unroll=4)
            def _(t):
                tv = t + zeros16
                for j in range(_D // 16):
                    v = rows[sbuf, t, pl.ds(j * 16, 16)]
                    plsc.store_scatter(outb.at[dbuf], [ciota[j], tv], v)

        for b in range(_NBUF - 1):
            build_list(b, b)
            g_start(b)

        def blk(i, carry):
            for u in range(4):
                b = i * 4 + u
                nxt = b + _NBUF - 1

                @pl.when(nxt < n)
                def _():
                    build_list(nxt, (u + _NBUF - 1) % _NBUF)
                    g_start((u + _NBUF - 1) % _NBUF)

                g_wait(u)

                @pl.when(b >= _WBUF)
                def _():
                    w_wait(b - _WBUF, u % _WBUF)

                tp_block(u, u % _WBUF)
                w_start(b, u % _WBUF)
            return carry

        lax.fori_loop(0, n // 4, blk, 0)
        w_wait(n - 2, (n - 2) % _WBUF)
        w_wait(n - 1, (n - 1) % _WBUF)

    return pl.kernel(
        body,
        mesh=plsc.VectorSubcoreMesh(core_axis_name="c", subcore_axis_name="s"),
        compiler_params=pltpu.CompilerParams(use_tc_tiling_on_sc=False,
                                             needs_layout_passes=False),
        out_type=jax.ShapeDtypeStruct((R, _D, T), jnp.float32),
        scratch_types=[
            pltpu.VMEM((T // _NW * R,), jnp.int32),
            pltpu.VMEM((_NBUF, _TB), jnp.int32),
            pltpu.VMEM((_NBUF, _TB, _D), jnp.float32),
            pltpu.VMEM((_WBUF, _D, _TB), jnp.float32),
            pltpu.SemaphoreType.DMA((_NBUF,)),
            pltpu.SemaphoreType.DMA((_WBUF,)),
        ],
    )


def kernel(token_ids, weight):
    T, R = token_ids.shape
    V = weight.shape[0]
    flat_ids = token_ids.reshape(-1).astype(jnp.int32)
    v_full = (V // _TB) * _TB
    wt_pad = jnp.pad(weight[v_full:].T, ((0, 0), (0, _TB - (V - v_full))))
    table_flat = _make_transpose(V)(weight.T, wt_pad)
    out3 = _make_lookup(T, R)(flat_ids, table_flat.reshape(V, _D))
    return jnp.transpose(out3, (2, 0, 1))
